# baseline (speedup 1.0000x reference)
import functools

import jax
import jax.numpy as jnp
from jax import lax
from jax.experimental import pallas as pl
from jax.experimental.pallas import tpu as pltpu
from jax.experimental.pallas import tpu_sc as plsc

_B = 4096
_CTX = 20
_D = 64
_OUT = 100000

_NC = 2
_NS = 16
_NW = _NC * _NS
_BPW = _B // _NW
_SUB = 16
_NSUB = _BPW // _SUB

_LANES = 16


def _sc_pool_body(idxh_hbm, par_hbm, table_hbm, pooled_hbm,
                  idxh_v, par_v, bufs, out_v, sem):
    wid = lax.axis_index("s") * _NC + lax.axis_index("c")
    base = wid * _BPW

    for sc in range(_NSUB):
        wbase = base + sc * _SUB
        pltpu.sync_copy(idxh_hbm.at[pl.ds(wbase, _SUB), :], idxh_v)
        pltpu.sync_copy(par_hbm.at[pl.ds(wbase, _SUB), :], par_v)
        descs = [
            pltpu.async_copy(
                table_hbm.at[idxh_v.at[j]], bufs.at[j], sem
            )
            for j in range(_SUB)
        ]
        for d in descs:
            d.wait()

        def reduce_one(j, carry):
            accs = [jnp.zeros((_LANES,), jnp.float32) for _ in range(_D // _LANES)]
            for c in range(_CTX):
                p16 = par_v[j, pl.ds(c * _LANES, _LANES)]
                for d in range(_D // _LANES):
                    left = bufs[j, c, pl.ds(d * _LANES, _LANES)]
                    right = bufs[j, c, pl.ds(_D + d * _LANES, _LANES)]
                    accs[d] = accs[d] + (left + (right - left) * p16)
            for d in range(_D // _LANES):
                out_v[j, pl.ds(d * _LANES, _LANES)] = accs[d]
            return carry

        lax.fori_loop(0, _SUB, reduce_one, 0)

        pltpu.sync_copy(out_v, pooled_hbm.at[pl.ds(wbase, _SUB), :])


def _sc_pool(idx_half, parity_b, table2):
    mesh = plsc.VectorSubcoreMesh(core_axis_name="c", subcore_axis_name="s")
    return pl.kernel(
        _sc_pool_body,
        out_type=jax.ShapeDtypeStruct((_B, _D), jnp.float32),
        mesh=mesh,
        scratch_types=[
            pltpu.VMEM((_SUB, _CTX), jnp.int32),
            pltpu.VMEM((_SUB, _CTX * _LANES), jnp.float32),
            pltpu.VMEM((_SUB, _CTX, 2 * _D), jnp.float32),
            pltpu.VMEM((_SUB, _D), jnp.float32),
            pltpu.SemaphoreType.DMA,
        ],
    )(idx_half, parity_b, table2)


_NT = 2048
_MT = 2048


def _mm_body(p_ref, w_ref, b_ref, o_ref):
    acc = lax.dot_general(
        p_ref[...], w_ref[...],
        (((1,), (1,)), ((), ())),
        preferred_element_type=jnp.float32,
    )
    o_ref[...] = acc + b_ref[...]


def _matmul(pooled, W, b):
    n_blocks = (_OUT + _NT - 1) // _NT
    n_pad = n_blocks * _NT
    W_p = jnp.pad(W, ((0, n_pad - _OUT), (0, 0)))
    b_p = jnp.pad(b, (0, n_pad - _OUT)).reshape(1, n_pad)
    return pl.pallas_call(
        _mm_body,
        grid=(_B // _MT, n_blocks),
        in_specs=[
            pl.BlockSpec((_MT, _D), lambda i, j: (i, 0)),
            pl.BlockSpec((_NT, _D), lambda i, j: (j, 0)),
            pl.BlockSpec((1, _NT), lambda i, j: (0, j)),
        ],
        out_specs=pl.BlockSpec((_MT, _NT), lambda i, j: (i, j)),
        out_shape=jax.ShapeDtypeStruct((_B, _OUT), jnp.float32),
    )(pooled, W_p, b_p)


def kernel(inputs, embed_table, W, b):
    idx = inputs.astype(jnp.int32)
    idx_half = idx >> 1
    parity = (idx & 1).astype(jnp.float32)
    parity_b = jnp.broadcast_to(
        parity[:, :, None], (_B, _CTX, _LANES)
    ).reshape(_B, _CTX * _LANES)
    table2 = embed_table.reshape(-1, 2 * _D)
    pooled = _sc_pool(idx_half, parity_b, table2)
    return _matmul(pooled, W, b)


# speedup vs baseline: 2.0749x; 2.0749x over previous
import functools

import jax
import jax.numpy as jnp
from jax import lax
from jax.experimental import pallas as pl
from jax.experimental.pallas import tpu as pltpu
from jax.experimental.pallas import tpu_sc as plsc

_B = 4096
_CTX = 20
_D = 64
_OUT = 100000

_NC = 2
_NS = 16
_NW = _NC * _NS
_BPW = _B // _NW
_SUB = 16
_NSUB = _BPW // _SUB

_LANES = 16


def _sc_pool_body(idxh_hbm, par_hbm, table_hbm, pooled_hbm,
                  idxh_v, par_v, bufs, out_v, sem):
    wid = lax.axis_index("s") * _NC + lax.axis_index("c")
    base = wid * _BPW

    for sc in range(_NSUB):
        wbase = base + sc * _SUB
        pltpu.sync_copy(idxh_hbm.at[pl.ds(wbase, _SUB), :], idxh_v)
        pltpu.sync_copy(par_hbm.at[pl.ds(wbase, _SUB), :], par_v)
        descs = [
            pltpu.async_copy(
                table_hbm.at[idxh_v.at[j]], bufs.at[j], sem
            )
            for j in range(_SUB)
        ]
        for d in descs:
            d.wait()

        def reduce_one(j, carry):
            accs = [jnp.zeros((_LANES,), jnp.float32) for _ in range(_D // _LANES)]
            for c in range(_CTX):
                p16 = par_v[j, pl.ds(c * _LANES, _LANES)]
                for d in range(_D // _LANES):
                    left = bufs[j, c, pl.ds(d * _LANES, _LANES)]
                    right = bufs[j, c, pl.ds(_D + d * _LANES, _LANES)]
                    accs[d] = accs[d] + (left + (right - left) * p16)
            for d in range(_D // _LANES):
                out_v[j, pl.ds(d * _LANES, _LANES)] = accs[d]
            return carry

        lax.fori_loop(0, _SUB, reduce_one, 0)

        pltpu.sync_copy(out_v, pooled_hbm.at[pl.ds(wbase, _SUB), :])


def _sc_pool(idx_half, parity_b, table2):
    mesh = plsc.VectorSubcoreMesh(core_axis_name="c", subcore_axis_name="s")
    return pl.kernel(
        _sc_pool_body,
        out_type=jax.ShapeDtypeStruct((_B, _D), jnp.float32),
        mesh=mesh,
        scratch_types=[
            pltpu.VMEM((_SUB, _CTX), jnp.int32),
            pltpu.VMEM((_SUB, _CTX * _LANES), jnp.float32),
            pltpu.VMEM((_SUB, _CTX, 2 * _D), jnp.float32),
            pltpu.VMEM((_SUB, _D), jnp.float32),
            pltpu.SemaphoreType.DMA,
        ],
    )(idx_half, parity_b, table2)


_NT = 512
_KA = 72


def _mm_body(w_ref, p_ref, o_ref):
    o_ref[...] = lax.dot_general(
        w_ref[...], p_ref[...],
        (((1,), (1,)), ((), ())),
        preferred_element_type=jnp.float32,
    )


def _matmul(pooled, W, b):
    n_blocks = (_OUT + _NT - 1) // _NT
    n_pad = n_blocks * _NT
    W_aug = jnp.concatenate(
        [W, b[:, None], jnp.zeros((_OUT, _KA - _D - 1), jnp.float32)], axis=1
    )
    W_aug = jnp.pad(W_aug, ((0, n_pad - _OUT), (0, 0)))
    p_aug = jnp.concatenate(
        [pooled, jnp.ones((_B, 1), jnp.float32),
         jnp.zeros((_B, _KA - _D - 1), jnp.float32)], axis=1
    )
    out_t = pl.pallas_call(
        _mm_body,
        grid=(n_blocks,),
        in_specs=[
            pl.BlockSpec((_NT, _KA), lambda j: (j, 0)),
            pl.BlockSpec((_B, _KA), lambda j: (0, 0)),
        ],
        out_specs=pl.BlockSpec((_NT, _B), lambda j: (j, 0)),
        out_shape=jax.ShapeDtypeStruct((_OUT, _B), jnp.float32),
    )(W_aug, p_aug)
    return out_t.T


def kernel(inputs, embed_table, W, b):
    idx = inputs.astype(jnp.int32)
    idx_half = idx >> 1
    parity = (idx & 1).astype(jnp.float32)
    parity_b = jnp.broadcast_to(
        parity[:, :, None], (_B, _CTX, _LANES)
    ).reshape(_B, _CTX * _LANES)
    table2 = embed_table.reshape(-1, 2 * _D)
    pooled = _sc_pool(idx_half, parity_b, table2)
    return _matmul(pooled, W, b)
